# baseline (speedup 1.0000x reference)
import jax
import jax.numpy as jnp
from jax.experimental import pallas as pl
from jax.experimental.pallas import tpu as pltpu

_BB = 40


def _body(nodes_ref, pooled_ref, keep_ref, out_ref):
    keep = keep_ref[...][:, :, None]
    out_ref[...] = nodes_ref[...] + pooled_ref[...] * keep


def kernel(nodes, mask, pooled_edges, locked_nodes):
    B, N, D = nodes.shape
    keepf = 1.0 - (mask & locked_nodes).astype(jnp.float32)
    bs3 = pl.BlockSpec((_BB, N, D), lambda i: (i, 0, 0))
    bsm = pl.BlockSpec((_BB, N), lambda i: (i, 0))
    return pl.pallas_call(
        _body,
        grid=(pl.cdiv(B, _BB),),
        in_specs=[bs3, bs3, bsm],
        out_specs=bs3,
        out_shape=jax.ShapeDtypeStruct((B, N, D), nodes.dtype),
        compiler_params=pltpu.CompilerParams(
            dimension_semantics=("parallel",),
        ),
    )(nodes, pooled_edges, keepf)
